# baseline (device time: 22742 ns/iter reference)
import jax
import jax.numpy as jnp
from jax import lax
from jax.experimental import pallas as pl
from jax.experimental.pallas import tpu as pltpu

N_DEV = 16
PLANE = 4
N_Z = 4
N_TOK = 1024
D_IN = 512
D_OUT = 1024
E_LOCAL = 4
ROWS = N_TOK // N_DEV
BLK = PLANE * ROWS
CAP1 = 16
CAP_ME = 128


def kernel(x, router_W, route_idx, expert_W, shared_W):
    def body(x_ref, rw_ref, ri_ref, ew_ref, sw_ref, out_ref,
             stage_ref, rp_ref, send_sems, recv_sems):
        d = lax.axis_index("i")

        barrier = pltpu.get_barrier_semaphore()
        for j in range(N_DEV):
            pl.semaphore_signal(barrier, inc=1, device_id=(j,),
                                device_id_type=pl.DeviceIdType.MESH)

        xv = x_ref[:, :]
        scores = xv @ rw_ref[:, :]
        m = jnp.max(scores, axis=-1, keepdims=True)
        p = jnp.exp(scores - m)
        probs = p / jnp.sum(p, axis=-1, keepdims=True)
        route = ri_ref[:, :]
        eids = lax.broadcasted_iota(route.dtype, scores.shape, 1)
        coef = jnp.sum(jnp.where(eids == route, probs, 0.0),
                       axis=-1, keepdims=True)

        ri_b = lax.broadcasted_iota(jnp.int32, (BLK, BLK), 0)
        ci_b = lax.broadcasted_iota(jnp.int32, (BLK, BLK), 1)
        tri_bd = jnp.logical_and(ci_b <= ri_b,
                                 lax.div(ci_b, ROWS) == lax.div(ri_b, ROWS)
                                 ).astype(jnp.float32)
        tri_lo = (ci_b <= ri_b).astype(jnp.float32)
        iota_col4 = lax.broadcasted_iota(jnp.int32, (BLK, PLANE * CAP1), 1)

        sends = []

        mine_all = lax.div(route, E_LOCAL) == d
        minef = mine_all.astype(jnp.float32)
        mine_mat = jnp.concatenate(
            [minef[b * BLK:(b + 1) * BLK] for b in range(N_Z)],
            axis=1)
        ranks = jnp.concatenate([tri_bd, tri_lo], axis=0) @ mine_mat
        cnts = ranks[2 * BLK - 1:2 * BLK, :]
        colg_blocks = []
        off = jnp.float32(0.0)
        for b in range(N_Z):
            colg_blocks.append(ranks[BLK:2 * BLK, b:b + 1] - 1.0 + off)
            off = off + cnts[0, b]
        colg = jnp.concatenate(colg_blocks, axis=0).astype(jnp.int32)
        pg = jnp.where(
            jnp.logical_and(mine_all, colg < CAP_ME),
            (colg == lax.broadcasted_iota(jnp.int32, (N_TOK, CAP_ME), 1)
             ).astype(jnp.float32),
            0.0)
        xg = lax.dot_general(pg, xv, (((0,), (0,)), ((), ())),
                             preferred_element_type=jnp.float32)
        cr = jnp.concatenate([coef, route.astype(jnp.float32)], axis=1)
        crg = lax.dot_general(pg, cr, (((0,), (0,)), ((), ())),
                              preferred_element_type=jnp.float32)
        coef_g = crg[:, 0:1]
        route_g = crg[:, 1:2]
        yg = (xg * jnp.where(route_g == d * E_LOCAL, coef_g, 0.0)) @ ew_ref[0]
        for e in range(1, E_LOCAL):
            yg += (xg * jnp.where(route_g == d * E_LOCAL + e, coef_g, 0.0)
                   ) @ ew_ref[e]

        pl.semaphore_wait(barrier, N_DEV)

        for zk in range(N_Z):
            rs = zk * BLK
            mine = mine_all[rs:rs + BLK]
            slots_i = (ranks[0:BLK, zk:zk + 1] - 1.0).astype(jnp.int32)
            colidx = (lax.broadcasted_iota(jnp.int32, (BLK, 1), 0)
                      // ROWS) * CAP1 + slots_i
            ptb = jnp.where(jnp.logical_and(mine, slots_i < CAP1),
                            (colidx == iota_col4).astype(jnp.float32),
                            0.0)
            colg_b = colg[rs:rs + BLK]
            pgb = jnp.where(
                jnp.logical_and(mine, colg_b < CAP_ME),
                (colg_b == lax.broadcasted_iota(
                    jnp.int32, (BLK, CAP_ME), 1)).astype(jnp.float32),
                0.0)
            compose = lax.dot_general(ptb, pgb, (((0,), (0,)), ((), ())),
                                      preferred_element_type=jnp.float32)
            g = compose @ yg
            stage_ref[pl.ds(zk * PLANE * CAP1, PLANE * CAP1), :] = (
                g.astype(jnp.bfloat16))

            for wp in range(PLANE):
                c = 4 * zk + wp
                rdma = pltpu.make_async_remote_copy(
                    src_ref=stage_ref.at[pl.ds(c * CAP1, CAP1), :],
                    dst_ref=rp_ref.at[pl.ds(d * CAP1, CAP1), :],
                    send_sem=send_sems.at[c],
                    recv_sem=recv_sems.at[d],
                    device_id=(c,),
                    device_id_type=pl.DeviceIdType.MESH,
                )
                rdma.start()
                sends.append(rdma)

        route_m = ri_ref[pl.ds(d * ROWS, ROWS), :]
        own_m = lax.div(route_m, E_LOCAL)
        onehot = (own_m == lax.broadcasted_iota(jnp.int32, (ROWS, N_DEV), 1)
                  ).astype(jnp.float32)
        tri64 = (lax.broadcasted_iota(jnp.int32, (ROWS, ROWS), 1)
                 <= lax.broadcasted_iota(jnp.int32, (ROWS, ROWS), 0)
                 ).astype(jnp.float32)
        counts = tri64 @ onehot
        slot_m = (jnp.sum(onehot * counts, axis=-1, keepdims=True)
                  ).astype(jnp.int32) - 1
        colidx_m = own_m * CAP1 + slot_m
        ptc = jnp.where(
            slot_m < CAP1,
            (colidx_m == lax.broadcasted_iota(
                jnp.int32, (ROWS, N_DEV * CAP1), 1)).astype(jnp.float32),
            0.0)

        shared = x_ref[pl.ds(d * ROWS, ROWS), :] @ sw_ref[:, :]

        for j in range(N_DEV):
            recv = pltpu.make_async_remote_copy(
                src_ref=stage_ref.at[pl.ds(0, CAP1), :],
                dst_ref=rp_ref.at[pl.ds(j * CAP1, CAP1), :],
                send_sem=send_sems.at[j],
                recv_sem=recv_sems.at[j],
                device_id=(d,),
                device_id_type=pl.DeviceIdType.MESH,
            )
            recv.wait_recv()
        out_ref[:, :] = shared + ptc @ rp_ref[:, :].astype(jnp.float32)

        for rdma in sends:
            rdma.wait_send()

    return pl.pallas_call(
        body,
        out_shape=jax.ShapeDtypeStruct((ROWS, D_OUT), jnp.float32),
        in_specs=[pl.BlockSpec(memory_space=pltpu.VMEM)] * 5,
        out_specs=pl.BlockSpec(memory_space=pltpu.VMEM),
        scratch_shapes=[
            pltpu.VMEM((N_DEV * CAP1, D_OUT), jnp.bfloat16),
            pltpu.VMEM((N_DEV * CAP1, D_OUT), jnp.bfloat16),
            pltpu.SemaphoreType.DMA((N_DEV,)),
            pltpu.SemaphoreType.DMA((N_DEV,)),
        ],
        compiler_params=pltpu.CompilerParams(collective_id=0),
    )(x, router_W, route_idx, expert_W, shared_W)
